# Initial kernel scaffold; baseline (speedup 1.0000x reference)
import jax
import jax.numpy as jnp
from jax.experimental import pallas as pl


def kernel(x, edge_index, W1, b1, W2, b2):
    raise NotImplementedError("write your pallas kernel here")


# speedup vs baseline: 1.8088x; 1.8088x over previous
import functools

import jax
import jax.numpy as jnp
from jax import lax
from jax.experimental import pallas as pl
from jax.experimental.pallas import tpu as pltpu
from jax.experimental.pallas import tpu_sc as plsc

N = 10000
E = 160000
C_IN = 128
C_HID = 128
C_OUT = 40
C2P = 64

K = 128
E_PAD = 163840
NCHUNKS = E_PAD // K
NW = 32
ROWS_PER_TEC = NCHUNKS // NW
N_COL = 10112
N_ACC = 10240
F1 = C_HID // NW
F2 = C2P // NW
IG = 40
NG = NCHUNKS // IG

_MESH = plsc.VectorSubcoreMesh(core_axis_name="c", subcore_axis_name="s")


def _sc_body(nf, with_deg, ht_hbm, src_hbm, dst_hbm, z_hbm, *refs):
    if with_deg:
        m_out, deg_out = refs[0], refs[1]
        scr = refs[2:]
    else:
        m_out, deg_out = refs[0], None
        scr = refs[1:]
    hloc = scr[:nf]
    acc = scr[nf:2 * nf]
    if with_deg:
        deg, sidx, didx, sem = scr[2 * nf:]
    else:
        sidx, didx, sem = scr[2 * nf:]
        deg = None

    cid = lax.axis_index("c")
    sid = lax.axis_index("s")
    wid = cid * 16 + sid

    for r in range(nf):
        pltpu.sync_copy(ht_hbm.at[wid * nf + r, 0], hloc[r])
        pltpu.sync_copy(z_hbm, acc[r])

    def grp_body(g, _):
        pltpu.sync_copy(src_hbm.at[pl.ds(g * IG, IG)], sidx)
        pltpu.sync_copy(dst_hbm.at[pl.ds(g * IG, IG)], didx)

        def vec_body(t, _):
            j = t // 8
            v = t % 8
            s16 = sidx[j, pl.ds(v * 16, 16)]
            d16 = didx[j, pl.ds(v * 16, 16)]
            for r in range(nf):
                vals = plsc.load_gather(hloc[r], [d16])
                plsc.addupdate_scatter(acc[r], [s16], vals)
            return 0

        lax.fori_loop(0, IG * 8, vec_body, 0)
        return 0

    lax.fori_loop(0, NG, grp_body, 0)

    for r in range(nf):
        pltpu.sync_copy(acc[r], m_out.at[wid * nf + r, 0])

    if with_deg:
        ones16 = jnp.ones((16,), jnp.float32)

        def dzero(i, _):
            deg[pl.ds(i * 16, 16)] = jnp.zeros((16,), jnp.float32)
            return 0

        lax.fori_loop(0, N_ACC // 16, dzero, 0)
        pltpu.sync_copy(src_hbm.at[pl.ds(wid * ROWS_PER_TEC, ROWS_PER_TEC)],
                        sidx)

        def dvec(t, _):
            s16 = sidx[t // 8, pl.ds((t % 8) * 16, 16)]
            plsc.addupdate_scatter(deg, [s16], ones16)
            return 0

        lax.fori_loop(0, ROWS_PER_TEC * 8, dvec, 0)
        pltpu.sync_copy(deg, deg_out.at[wid, 0])


@jax.jit
def _sc_layer1(ht3, src2d, dst2d, zrow):
    body = functools.partial(_sc_body, F1, True)
    return pl.kernel(
        body,
        out_type=(
            jax.ShapeDtypeStruct((NW * F1, 1, N_ACC), jnp.float32),
            jax.ShapeDtypeStruct((NW, 1, N_ACC), jnp.float32),
        ),
        mesh=_MESH,
        compiler_params=pltpu.CompilerParams(needs_layout_passes=False),
        scratch_types=(
            *[pltpu.VMEM((N_COL,), jnp.float32) for _ in range(F1)],
            *[pltpu.VMEM((N_ACC,), jnp.float32) for _ in range(F1)],
            pltpu.VMEM((N_ACC,), jnp.float32),
            pltpu.VMEM((IG, K), jnp.int32),
            pltpu.VMEM((IG, K), jnp.int32),
            pltpu.SemaphoreType.DMA,
        ),
    )(ht3, src2d, dst2d, zrow)


@jax.jit
def _sc_layer2(ht3, src2d, dst2d, zrow):
    body = functools.partial(_sc_body, F2, False)
    return pl.kernel(
        body,
        out_type=jax.ShapeDtypeStruct((NW * F2, 1, N_ACC), jnp.float32),
        mesh=_MESH,
        compiler_params=pltpu.CompilerParams(needs_layout_passes=False),
        scratch_types=(
            *[pltpu.VMEM((N_ACC,), jnp.float32) for _ in range(F2)],
            *[pltpu.VMEM((N_ACC,), jnp.float32) for _ in range(F2)],
            pltpu.VMEM((IG, K), jnp.int32),
            pltpu.VMEM((IG, K), jnp.int32),
            pltpu.SemaphoreType.DMA,
        ),
    )(ht3, src2d, dst2d, zrow)


_BLK = 1024


def _tc1_body(x_ref, w_ref, b_ref, o_ref):
    o_ref[...] = (
        jnp.dot(x_ref[...], w_ref[...], preferred_element_type=jnp.float32)
        + b_ref[...]
    )


def _tc1(x, W1, b1):
    return pl.pallas_call(
        _tc1_body,
        grid=(10,),
        in_specs=[
            pl.BlockSpec((1000, C_IN), lambda i: (i, 0)),
            pl.BlockSpec((C_IN, C_HID), lambda i: (0, 0)),
            pl.BlockSpec((1, C_HID), lambda i: (0, 0)),
        ],
        out_specs=pl.BlockSpec((1000, C_HID), lambda i: (i, 0)),
        out_shape=jax.ShapeDtypeStruct((N, C_HID), jnp.float32),
    )(x, W1, b1.reshape(1, C_HID))


def _tc2_body(m_ref, d_ref, w_ref, b_ref, h_ref, inv_ref):
    deg = jnp.sum(d_ref[...], axis=1, keepdims=True)
    inv = 1.0 / deg
    t = jnp.maximum(m_ref[...] * inv, 0.0)
    h_ref[...] = (
        jnp.dot(t, w_ref[...], preferred_element_type=jnp.float32) + b_ref[...]
    )
    inv_ref[...] = inv


def _tc2(m1, degT, W2p, b2p):
    return pl.pallas_call(
        _tc2_body,
        grid=(N_ACC // _BLK,),
        in_specs=[
            pl.BlockSpec((_BLK, C_HID), lambda i: (i, 0)),
            pl.BlockSpec((_BLK, NW), lambda i: (i, 0)),
            pl.BlockSpec((C_HID, C2P), lambda i: (0, 0)),
            pl.BlockSpec((1, C2P), lambda i: (0, 0)),
        ],
        out_specs=[
            pl.BlockSpec((_BLK, C2P), lambda i: (i, 0)),
            pl.BlockSpec((_BLK, 1), lambda i: (i, 0)),
        ],
        out_shape=[
            jax.ShapeDtypeStruct((N_ACC, C2P), jnp.float32),
            jax.ShapeDtypeStruct((N_ACC, 1), jnp.float32),
        ],
    )(m1, degT, W2p, b2p.reshape(1, C2P))


def _tc3_body(m_ref, inv_ref, o_ref):
    o_ref[...] = (m_ref[...] * inv_ref[...])[:, :C_OUT]


def _tc3(m2, inv):
    return pl.pallas_call(
        _tc3_body,
        grid=(N_ACC // _BLK,),
        in_specs=[
            pl.BlockSpec((_BLK, C2P), lambda i: (i, 0)),
            pl.BlockSpec((_BLK, 1), lambda i: (i, 0)),
        ],
        out_specs=pl.BlockSpec((_BLK, C_OUT), lambda i: (i, 0)),
        out_shape=jax.ShapeDtypeStruct((N_ACC, C_OUT), jnp.float32),
    )(m2, inv)


def kernel(x, edge_index, W1, b1, W2, b2):
    pad_src = jnp.full((E_PAD - E,), N, jnp.int32)
    pad_dst = jnp.zeros((E_PAD - E,), jnp.int32)
    src2d = jnp.concatenate([edge_index[0], pad_src]).reshape(NCHUNKS, K)
    dst2d = jnp.concatenate([edge_index[1], pad_dst]).reshape(NCHUNKS, K)
    W2p = jnp.pad(W2, ((0, 0), (0, C2P - C_OUT)))
    b2p = jnp.pad(b2, (0, C2P - C_OUT))
    zrow = jnp.zeros((N_ACC,), jnp.float32)

    h1p = _tc1(x, W1, b1)
    h1t3 = jnp.pad(h1p.T, ((0, 0), (0, N_COL - N))).reshape(C_HID, 1, N_COL)
    m1t3, degp = _sc_layer1(h1t3, src2d, dst2d, zrow)
    m1 = m1t3.reshape(C_HID, N_ACC).T
    degT = degp.reshape(NW, N_ACC).T
    h2p, inv = _tc2(m1, degT, W2p, b2p)
    h2t3 = h2p.T.reshape(C2P, 1, N_ACC)
    m2t3 = _sc_layer2(h2t3, src2d, dst2d, zrow)
    m2 = m2t3.reshape(C2P, N_ACC).T
    out = _tc3(m2, inv)
    return out[:N]


# speedup vs baseline: 1.8126x; 1.0021x over previous
import functools

import jax
import jax.numpy as jnp
from jax import lax
from jax.experimental import pallas as pl
from jax.experimental.pallas import tpu as pltpu
from jax.experimental.pallas import tpu_sc as plsc

N = 10000
E = 160000
C_IN = 128
C_HID = 128
C_OUT = 40
C2P = 64

K = 128
E_PAD = 163840
NCHUNKS = E_PAD // K
NW = 32
ROWS_PER_TEC = NCHUNKS // NW
N_COL = 10112
N_ACC = 10240
F1 = C_HID // NW
F2 = C2P // NW
IG = 40
NG = NCHUNKS // IG

_MESH = plsc.VectorSubcoreMesh(core_axis_name="c", subcore_axis_name="s")


def _sc_body(nf, with_deg, ht_hbm, src_hbm, dst_hbm, z_hbm, *refs):
    if with_deg:
        m_out, deg_out = refs[0], refs[1]
        scr = refs[2:]
    else:
        m_out, deg_out = refs[0], None
        scr = refs[1:]
    hloc = scr[:nf]
    acc = scr[nf:2 * nf]
    if with_deg:
        deg, sidx, didx, sem = scr[2 * nf:]
    else:
        sidx, didx, sem = scr[2 * nf:]
        deg = None

    cid = lax.axis_index("c")
    sid = lax.axis_index("s")
    wid = cid * 16 + sid

    for r in range(nf):
        pltpu.sync_copy(ht_hbm.at[wid * nf + r, 0], hloc[r])
        pltpu.sync_copy(z_hbm, acc[r])

    def grp_body(g, _):
        pltpu.sync_copy(src_hbm.at[pl.ds(g * IG, IG)], sidx)
        pltpu.sync_copy(dst_hbm.at[pl.ds(g * IG, IG)], didx)

        def row_body(j, _):
            for v in range(8):
                s16 = sidx[j, pl.ds(v * 16, 16)]
                d16 = didx[j, pl.ds(v * 16, 16)]
                for r in range(nf):
                    vals = plsc.load_gather(hloc[r], [d16])
                    plsc.addupdate_scatter(acc[r], [s16], vals)
            return 0

        lax.fori_loop(0, IG, row_body, 0)
        return 0

    lax.fori_loop(0, NG, grp_body, 0)

    for r in range(nf):
        pltpu.sync_copy(acc[r], m_out.at[wid * nf + r, 0])

    if with_deg:
        ones16 = jnp.ones((16,), jnp.float32)

        def dzero(i, _):
            deg[pl.ds(i * 16, 16)] = jnp.zeros((16,), jnp.float32)
            return 0

        lax.fori_loop(0, N_ACC // 16, dzero, 0)
        pltpu.sync_copy(src_hbm.at[pl.ds(wid * ROWS_PER_TEC, ROWS_PER_TEC)],
                        sidx)

        def dvec(j, _):
            for v in range(8):
                s16 = sidx[j, pl.ds(v * 16, 16)]
                plsc.addupdate_scatter(deg, [s16], ones16)
            return 0

        lax.fori_loop(0, ROWS_PER_TEC, dvec, 0)
        pltpu.sync_copy(deg, deg_out.at[wid, 0])


@jax.jit
def _sc_layer1(ht3, src2d, dst2d, zrow):
    body = functools.partial(_sc_body, F1, True)
    return pl.kernel(
        body,
        out_type=(
            jax.ShapeDtypeStruct((NW * F1, 1, N_ACC), jnp.float32),
            jax.ShapeDtypeStruct((NW, 1, N_ACC), jnp.float32),
        ),
        mesh=_MESH,
        compiler_params=pltpu.CompilerParams(needs_layout_passes=False),
        scratch_types=(
            *[pltpu.VMEM((N_COL,), jnp.float32) for _ in range(F1)],
            *[pltpu.VMEM((N_ACC,), jnp.float32) for _ in range(F1)],
            pltpu.VMEM((N_ACC,), jnp.float32),
            pltpu.VMEM((IG, K), jnp.int32),
            pltpu.VMEM((IG, K), jnp.int32),
            pltpu.SemaphoreType.DMA,
        ),
    )(ht3, src2d, dst2d, zrow)


@jax.jit
def _sc_layer2(ht3, src2d, dst2d, zrow):
    body = functools.partial(_sc_body, F2, False)
    return pl.kernel(
        body,
        out_type=jax.ShapeDtypeStruct((NW * F2, 1, N_ACC), jnp.float32),
        mesh=_MESH,
        compiler_params=pltpu.CompilerParams(needs_layout_passes=False),
        scratch_types=(
            *[pltpu.VMEM((N_ACC,), jnp.float32) for _ in range(F2)],
            *[pltpu.VMEM((N_ACC,), jnp.float32) for _ in range(F2)],
            pltpu.VMEM((IG, K), jnp.int32),
            pltpu.VMEM((IG, K), jnp.int32),
            pltpu.SemaphoreType.DMA,
        ),
    )(ht3, src2d, dst2d, zrow)


_BLK = 1024


def _tc1_body(x_ref, w_ref, b_ref, o_ref):
    o_ref[...] = (
        jnp.dot(x_ref[...], w_ref[...], preferred_element_type=jnp.float32)
        + b_ref[...]
    )


def _tc1(x, W1, b1):
    return pl.pallas_call(
        _tc1_body,
        grid=(10,),
        in_specs=[
            pl.BlockSpec((1000, C_IN), lambda i: (i, 0)),
            pl.BlockSpec((C_IN, C_HID), lambda i: (0, 0)),
            pl.BlockSpec((1, C_HID), lambda i: (0, 0)),
        ],
        out_specs=pl.BlockSpec((1000, C_HID), lambda i: (i, 0)),
        out_shape=jax.ShapeDtypeStruct((N, C_HID), jnp.float32),
    )(x, W1, b1.reshape(1, C_HID))


def _tc2_body(m_ref, d_ref, w_ref, b_ref, h_ref, inv_ref):
    deg = jnp.sum(d_ref[...], axis=1, keepdims=True)
    inv = 1.0 / deg
    t = jnp.maximum(m_ref[...] * inv, 0.0)
    h_ref[...] = (
        jnp.dot(t, w_ref[...], preferred_element_type=jnp.float32) + b_ref[...]
    )
    inv_ref[...] = inv


def _tc2(m1, degT, W2p, b2p):
    return pl.pallas_call(
        _tc2_body,
        grid=(N_ACC // _BLK,),
        in_specs=[
            pl.BlockSpec((_BLK, C_HID), lambda i: (i, 0)),
            pl.BlockSpec((_BLK, NW), lambda i: (i, 0)),
            pl.BlockSpec((C_HID, C2P), lambda i: (0, 0)),
            pl.BlockSpec((1, C2P), lambda i: (0, 0)),
        ],
        out_specs=[
            pl.BlockSpec((_BLK, C2P), lambda i: (i, 0)),
            pl.BlockSpec((_BLK, 1), lambda i: (i, 0)),
        ],
        out_shape=[
            jax.ShapeDtypeStruct((N_ACC, C2P), jnp.float32),
            jax.ShapeDtypeStruct((N_ACC, 1), jnp.float32),
        ],
    )(m1, degT, W2p, b2p.reshape(1, C2P))


def _tc3_body(m_ref, inv_ref, o_ref):
    o_ref[...] = (m_ref[...] * inv_ref[...])[:, :C_OUT]


def _tc3(m2, inv):
    return pl.pallas_call(
        _tc3_body,
        grid=(N_ACC // _BLK,),
        in_specs=[
            pl.BlockSpec((_BLK, C2P), lambda i: (i, 0)),
            pl.BlockSpec((_BLK, 1), lambda i: (i, 0)),
        ],
        out_specs=pl.BlockSpec((_BLK, C_OUT), lambda i: (i, 0)),
        out_shape=jax.ShapeDtypeStruct((N_ACC, C_OUT), jnp.float32),
    )(m2, inv)


def kernel(x, edge_index, W1, b1, W2, b2):
    pad_src = jnp.full((E_PAD - E,), N, jnp.int32)
    pad_dst = jnp.zeros((E_PAD - E,), jnp.int32)
    src2d = jnp.concatenate([edge_index[0], pad_src]).reshape(NCHUNKS, K)
    dst2d = jnp.concatenate([edge_index[1], pad_dst]).reshape(NCHUNKS, K)
    W2p = jnp.pad(W2, ((0, 0), (0, C2P - C_OUT)))
    b2p = jnp.pad(b2, (0, C2P - C_OUT))
    zrow = jnp.zeros((N_ACC,), jnp.float32)

    h1p = _tc1(x, W1, b1)
    h1t3 = jnp.pad(h1p.T, ((0, 0), (0, N_COL - N))).reshape(C_HID, 1, N_COL)
    m1t3, degp = _sc_layer1(h1t3, src2d, dst2d, zrow)
    m1 = m1t3.reshape(C_HID, N_ACC).T
    degT = degp.reshape(NW, N_ACC).T
    h2p, inv = _tc2(m1, degT, W2p, b2p)
    h2t3 = h2p.T.reshape(C2P, 1, N_ACC)
    m2t3 = _sc_layer2(h2t3, src2d, dst2d, zrow)
    m2 = m2t3.reshape(C2P, N_ACC).T
    out = _tc3(m2, inv)
    return out[:N]
